# Initial kernel scaffold; baseline (speedup 1.0000x reference)
import functools

import jax
import jax.numpy as jnp
from jax.experimental import pallas as pl

F32 = jnp.float32


def _mish(x):
    sp = jnp.where(x > 20.0, x, jnp.log1p(jnp.exp(jnp.minimum(x, 20.0))))
    return x * jnp.tanh(sp)


def _tree_body(x_ref, idx_ref, w1, b1r, w2, b2r, w3, b3r, w4, b4r, out_ref):
    x = x_ref[0]
    ids = idx_ref[0]
    T = x.shape[1]
    iota_jt = jax.lax.broadcasted_iota(jnp.int32, (T, T), 0)
    col0 = jax.lax.broadcasted_iota(jnp.int32, (1, T), 1) == 0

    for (wr, br) in ((w1, b1r), (w2, b2r), (w3, b3r), (w4, b4r)):
        w = wr[...]
        acc = None
        for k in range(3):
            y = jnp.dot(w[k], x, preferred_element_type=F32)
            g = (iota_jt == ids[k:k + 1, :]).astype(F32)
            t = jnp.dot(y, g, preferred_element_type=F32)
            acc = t if acc is None else acc + t
        out = acc + br[...]
        out = jnp.where(col0, 0.0, out)
        n = out.size
        m = jnp.mean(out)
        ss = jnp.sum((out - m) ** 2)
        std = jnp.sqrt(ss / (n - 1))
        x = _mish((out - m) / (std + 1e-5))
    out_ref[0, :] = jnp.max(x, axis=1)


def _final_body(p_ref, wt_ref, lb_ref, g_ref, b_ref, out_ref):
    y0 = jnp.dot(p_ref[...], wt_ref[...], preferred_element_type=F32) + lb_ref[...]
    mean = jnp.mean(y0, axis=0, keepdims=True)
    var = jnp.mean((y0 - mean) ** 2, axis=0, keepdims=True)
    out_ref[...] = (y0 - mean) / jnp.sqrt(var + 1e-5) * g_ref[...] + b_ref[...]


@jax.jit
def kernel(trees_data, trees_indexes, W1, b1, W2, b2, W3, b3, W4, b4, lin_W, lin_b, bn_g, bn_b):
    B, C0, T = trees_data.shape

    idx = trees_indexes.reshape(B, T - 1, 3).transpose(0, 2, 1).astype(jnp.int32)
    sent = jnp.full((B, 3, 1), T + 7, jnp.int32)
    idxp = jnp.concatenate([sent, idx], axis=2)

    ws = [jnp.transpose(W, (2, 0, 1)) for W in (W1, W2, W3, W4)]
    bs = [b.reshape(-1, 1) for b in (b1, b2, b3, b4)]

    def full(s):
        return pl.BlockSpec(s, lambda *_: (0,) * len(s))

    in_specs = [
        pl.BlockSpec((1, C0, T), lambda i: (i, 0, 0)),
        pl.BlockSpec((1, 3, T), lambda i: (i, 0, 0)),
    ]
    for w, b in zip(ws, bs):
        in_specs.append(full(w.shape))
        in_specs.append(full(b.shape))

    pooled = pl.pallas_call(
        _tree_body,
        grid=(B,),
        in_specs=in_specs,
        out_specs=pl.BlockSpec((1, ws[-1].shape[1]), lambda i: (i, 0)),
        out_shape=jax.ShapeDtypeStruct((B, ws[-1].shape[1]), F32),
    )(trees_data, idxp, ws[0], bs[0], ws[1], bs[1], ws[2], bs[2], ws[3], bs[3])

    Z = lin_W.shape[0]
    y = pl.pallas_call(
        _final_body,
        in_specs=[
            pl.BlockSpec(pooled.shape, lambda: (0, 0)),
            pl.BlockSpec((lin_W.shape[1], Z), lambda: (0, 0)),
            pl.BlockSpec((1, Z), lambda: (0, 0)),
            pl.BlockSpec((1, Z), lambda: (0, 0)),
            pl.BlockSpec((1, Z), lambda: (0, 0)),
        ],
        out_specs=pl.BlockSpec((B, Z), lambda: (0, 0)),
        out_shape=jax.ShapeDtypeStruct((B, Z), F32),
    )(pooled, lin_W.T, lin_b.reshape(1, -1), bn_g.reshape(1, -1), bn_b.reshape(1, -1))

    return (y, trees_indexes)


# speedup vs baseline: 2188.6943x; 2188.6943x over previous
import functools

import jax
import jax.numpy as jnp
from jax.experimental import pallas as pl

F32 = jnp.float32


def _mish(x):
    sp = jnp.where(x > 20.0, x, jnp.log1p(jnp.exp(jnp.minimum(x, 20.0))))
    return x * jnp.tanh(sp)


def _tree_body(x_ref, idx_ref, w1, b1r, w2, b2r, w3, b3r, w4, b4r, out_ref):
    x = x_ref[0]
    ids = idx_ref[0]
    T = x.shape[1]
    iota_jt = jax.lax.broadcasted_iota(jnp.int32, (T, T), 0)
    col0 = jax.lax.broadcasted_iota(jnp.int32, (1, T), 1) == 0

    for (wr, br) in ((w1, b1r), (w2, b2r), (w3, b3r), (w4, b4r)):
        w = wr[...]
        xb = x.astype(jnp.bfloat16)
        acc = None
        for k in range(3):
            y = jnp.dot(w[k], xb, preferred_element_type=F32)
            g = (iota_jt == ids[k:k + 1, :]).astype(jnp.bfloat16)
            y_hi = y.astype(jnp.bfloat16)
            y_lo = (y - y_hi.astype(F32)).astype(jnp.bfloat16)
            t = (jnp.dot(y_hi, g, preferred_element_type=F32)
                 + jnp.dot(y_lo, g, preferred_element_type=F32))
            acc = t if acc is None else acc + t
        out = acc + br[...]
        out = jnp.where(col0, 0.0, out)
        n = out.size
        m = jnp.mean(out)
        ss = jnp.sum((out - m) ** 2)
        std = jnp.sqrt(ss / (n - 1))
        x = _mish((out - m) / (std + 1e-5))
    out_ref[0, 0, :] = jnp.max(x, axis=1)


def _final_body(p_ref, wt_ref, lb_ref, g_ref, b_ref, out_ref):
    y0 = jnp.dot(p_ref[...].astype(jnp.bfloat16), wt_ref[...].astype(jnp.bfloat16),
                 preferred_element_type=F32) + lb_ref[...]
    mean = jnp.mean(y0, axis=0, keepdims=True)
    var = jnp.mean((y0 - mean) ** 2, axis=0, keepdims=True)
    out_ref[...] = (y0 - mean) / jnp.sqrt(var + 1e-5) * g_ref[...] + b_ref[...]


@jax.jit
def kernel(trees_data, trees_indexes, W1, b1, W2, b2, W3, b3, W4, b4, lin_W, lin_b, bn_g, bn_b):
    B, C0, T = trees_data.shape

    idx = trees_indexes.reshape(B, T - 1, 3).transpose(0, 2, 1).astype(jnp.int32)
    sent = jnp.full((B, 3, 1), T + 7, jnp.int32)
    idxp = jnp.concatenate([sent, idx], axis=2)

    ws = [jnp.transpose(W, (2, 0, 1)).astype(jnp.bfloat16) for W in (W1, W2, W3, W4)]
    bs = [b.reshape(-1, 1) for b in (b1, b2, b3, b4)]

    def full(s):
        return pl.BlockSpec(s, lambda *_: (0,) * len(s))

    in_specs = [
        pl.BlockSpec((1, C0, T), lambda i: (i, 0, 0)),
        pl.BlockSpec((1, 3, T), lambda i: (i, 0, 0)),
    ]
    for w, b in zip(ws, bs):
        in_specs.append(full(w.shape))
        in_specs.append(full(b.shape))

    pooled = pl.pallas_call(
        _tree_body,
        grid=(B,),
        in_specs=in_specs,
        out_specs=pl.BlockSpec((1, 1, ws[-1].shape[1]), lambda i: (i, 0, 0)),
        out_shape=jax.ShapeDtypeStruct((B, 1, ws[-1].shape[1]), F32),
    )(trees_data, idxp, ws[0], bs[0], ws[1], bs[1], ws[2], bs[2], ws[3], bs[3])
    pooled = pooled.reshape(B, ws[-1].shape[1])

    Z = lin_W.shape[0]
    y = pl.pallas_call(
        _final_body,
        in_specs=[
            pl.BlockSpec(pooled.shape, lambda: (0, 0)),
            pl.BlockSpec((lin_W.shape[1], Z), lambda: (0, 0)),
            pl.BlockSpec((1, Z), lambda: (0, 0)),
            pl.BlockSpec((1, Z), lambda: (0, 0)),
            pl.BlockSpec((1, Z), lambda: (0, 0)),
        ],
        out_specs=pl.BlockSpec((B, Z), lambda: (0, 0)),
        out_shape=jax.ShapeDtypeStruct((B, Z), F32),
    )(pooled, lin_W.T, lin_b.reshape(1, -1), bn_g.reshape(1, -1), bn_b.reshape(1, -1))

    return (y, trees_indexes)
